# baseline (speedup 1.0000x reference)
import jax
import jax.numpy as jnp
import numpy as np
from jax import lax
from jax.experimental import pallas as pl
from jax.experimental.pallas import tpu as pltpu
from jax.experimental.pallas import tpu_sc as plsc

N = 10000
E = 320000
D = 128

NC = 2
NS = 16
NW = NC * NS
EPW = E // NW
CHUNK = 40
NCHUNK = EPW // CHUNK
RCH = N // CHUNK

_PERM = np.zeros(D, np.int32)
for _t in range(D // 32):
    for _k in range(16):
        _PERM[32 * _t + _k] = 32 * _t + 2 * _k
        _PERM[32 * _t + 16 + _k] = 32 * _t + 2 * _k + 1


def _scale_chunk(rows_bf, rows_v, adj_v, d):
    for k in range(CHUNK // 8):
        base = min(8 * k, CHUNK - 16)
        loff = 8 * k - base
        a16 = adj_v[d, pl.ds(base, 16)]
        scaled = []
        for i in range(8):
            e = 8 * k + i
            a = jnp.full((16,), a16[loff + i], jnp.float32)
            for j in range(D // 32):
                v = rows_bf[d, e, pl.ds(16 * j, 16)]
                lo = jax.lax.bitcast_convert_type(v << 16, jnp.float32)
                hi = jax.lax.bitcast_convert_type(v & jnp.int32(-65536), jnp.float32)
                scaled.append(lo * a)
                scaled.append(hi * a)
        for i in range(8):
            e = 8 * k + i
            for j in range(D // 16):
                rows_v[d, e, pl.ds(j * 16, 16)] = scaled[i * (D // 16) + j]


def _sc_body(x_hbm, dst_hbm, src_hbm, adj_hbm, out_hbm,
             src_v, dst_v, adj_v, rows_bf, rows_v, acc,
             gs0, gs1, is0, is1, ss0, ss1):
    cid = lax.axis_index("c")
    sid = lax.axis_index("s")
    gsem = (gs0, gs1)
    isem = (is0, is1)
    ssem = (ss0, ss1)

    zero16 = jnp.zeros((16,), jnp.float32)

    def zrow(i, carry):
        for j in range(D // 16):
            rows_v[0, i, pl.ds(j * 16, 16)] = zero16
        return carry

    lax.fori_loop(0, CHUNK, zrow, 0)
    zbuf = rows_v.at[0]
    for k in range(-(-RCH // NS)):
        c = sid + NS * k

        @pl.when(c < RCH)
        def _():
            pltpu.sync_copy(zbuf, acc.at[pl.ds(c * CHUNK, CHUNK)])
    plsc.subcore_barrier()

    wid = sid * NC + cid

    def fetch_idx(g, d, sem):
        pltpu.async_copy(src_hbm.at[wid, g], src_v.at[d], sem)
        pltpu.async_copy(dst_hbm.at[wid, g], dst_v.at[d], sem)
        pltpu.async_copy(adj_hbm.at[wid, g], adj_v.at[d], sem)

    def wait_idx(g, d, sem):
        pltpu.make_async_copy(src_hbm.at[wid, g], src_v.at[d], sem).wait()
        pltpu.make_async_copy(dst_hbm.at[wid, g], dst_v.at[d], sem).wait()
        pltpu.make_async_copy(adj_hbm.at[wid, g], adj_v.at[d], sem).wait()

    def fire_gather(d, sem):
        pltpu.async_copy(x_hbm.at[src_v.at[d]], rows_bf.at[d], sem)

    def drain_gather(d, sem):
        pltpu.make_async_copy(x_hbm.at[src_v.at[d]], rows_bf.at[d], sem).wait()

    def drain_scatter(d, sem):
        pltpu.make_async_copy(rows_v.at[d], acc.at[dst_v.at[d]], sem).wait()

    fetch_idx(0, 0, isem[0])
    wait_idx(0, 0, isem[0])
    fire_gather(0, gsem[0])
    fetch_idx(1, 1, isem[1])

    def dbl_body(gg, carry):
        for d in range(2):
            g = gg * 2 + d
            drain_gather(d, gsem[d])

            @pl.when(g + 1 < NCHUNK)
            def _():
                wait_idx(g + 1, 1 - d, isem[1 - d])
                fire_gather(1 - d, gsem[1 - d])

            _scale_chunk(rows_bf, rows_v, adj_v, d)
            pltpu.async_copy(rows_v.at[d], acc.at[dst_v.at[d]],
                             ssem[d], add=True)
            drain_scatter(d, ssem[d])

            @pl.when(g + 2 < NCHUNK)
            def _():
                fetch_idx(g + 2, d, isem[d])
        return carry

    lax.fori_loop(0, NCHUNK // 2, dbl_body, 0)
    plsc.subcore_barrier()

    for k in range(-(-RCH // NS)):
        c = sid + NS * k

        @pl.when(c < RCH)
        def _():
            rs = pl.ds(c * CHUNK, CHUNK)
            pltpu.sync_copy(acc.at[rs], zbuf)
            pltpu.sync_copy(zbuf, out_hbm.at[cid, rs])


@jax.jit
def _sc_spmm(x, dst, src, adj_vals):
    mesh = plsc.VectorSubcoreMesh(core_axis_name="c", subcore_axis_name="s")
    return pl.kernel(
        _sc_body,
        out_type=jax.ShapeDtypeStruct((NC, N, D), jnp.float32),
        mesh=mesh,
        compiler_params=pltpu.CompilerParams(use_tc_tiling_on_sc=False),
        scratch_types=[
            pltpu.VMEM((2, CHUNK), jnp.int32),
            pltpu.VMEM((2, CHUNK), jnp.int32),
            pltpu.VMEM((2, CHUNK), jnp.float32),
            pltpu.VMEM((2, CHUNK, D // 2), jnp.int32),
            pltpu.VMEM((2, CHUNK, D), jnp.float32),
            pltpu.VMEM_SHARED((N, D), jnp.float32),
            pltpu.SemaphoreType.DMA,
            pltpu.SemaphoreType.DMA,
            pltpu.SemaphoreType.DMA,
            pltpu.SemaphoreType.DMA,
            pltpu.SemaphoreType.DMA,
            pltpu.SemaphoreType.DMA,
        ],
    )(x, dst.reshape(NW, NCHUNK, CHUNK),
      src.reshape(NW, NCHUNK, CHUNK),
      adj_vals.reshape(NW, NCHUNK, CHUNK))


def _tc_body(p_ref, w_ref, b_ref, o_ref):
    s = p_ref[0] + p_ref[1]
    acc = jnp.dot(s, w_ref[...], preferred_element_type=jnp.float32)
    o_ref[...] = jnp.maximum(acc + b_ref[...], 0.0)


BM = 1000


@jax.jit
def _tc_epilogue(partials, W, b):
    return pl.pallas_call(
        _tc_body,
        grid=(N // BM,),
        in_specs=[
            pl.BlockSpec((NC, BM, D), lambda i: (0, i, 0)),
            pl.BlockSpec((D, D), lambda i: (0, 0)),
            pl.BlockSpec((1, D), lambda i: (0, 0)),
        ],
        out_specs=pl.BlockSpec((BM, D), lambda i: (i, 0)),
        out_shape=jax.ShapeDtypeStruct((N, D), jnp.float32),
    )(partials, W, b.reshape(1, D))


def kernel(x, edge_index, adj_vals, W, b):
    xbf = x.astype(jnp.bfloat16).reshape(N, D // 2, 2)
    xpk = jax.lax.bitcast_convert_type(xbf, jnp.int32)
    dst = edge_index[0]
    src = edge_index[1]
    partials = _sc_spmm(xpk, dst, src, adj_vals)
    return _tc_epilogue(partials, W.astype(jnp.float32)[_PERM], b)


# speedup vs baseline: 1.1472x; 1.1472x over previous
import jax
import jax.numpy as jnp
from jax import lax
from jax.experimental import pallas as pl
from jax.experimental.pallas import tpu as pltpu
from jax.experimental.pallas import tpu_sc as plsc

N = 10000
E = 320000
D = 128

NC = 2
NS = 16
DH = D // NC
EPW = E // NS
CHUNK = 80
NBUF = 5
BLK = NBUF * CHUNK
NOUT = EPW // BLK
RCH = N // CHUNK


def _scale_chunk(rows_v, adj_v, d, b):
    for k in range(CHUNK // 8):
        base = min(8 * k, CHUNK - 16)
        loff = 8 * k - base
        a16 = adj_v[d, b, pl.ds(base, 16)]
        scaled = []
        for i in range(8):
            e = 8 * k + i
            a = jnp.full((16,), a16[loff + i], jnp.float32)
            for j in range(DH // 16):
                scaled.append(rows_v[d, b, e, pl.ds(j * 16, 16)] * a)
        for i in range(8):
            e = 8 * k + i
            for j in range(DH // 16):
                rows_v[d, b, e, pl.ds(j * 16, 16)] = scaled[i * (DH // 16) + j]


def _sc_body(h_hbm, dst_hbm, src_hbm, adj_hbm, b_hbm, out_hbm,
             src_v, dst_v, adj_v, rows_v, bvec, acc,
             gs0, gs1, is0, is1, ss0, ss1):
    cid = lax.axis_index("c")
    sid = lax.axis_index("s")
    gsem = (gs0, gs1)
    isem = (is0, is1)
    ssem = (ss0, ss1)

    zero16 = jnp.zeros((16,), jnp.float32)

    def zrow(i, carry):
        for j in range(DH // 16):
            rows_v[0, 0, i, pl.ds(j * 16, 16)] = zero16
        return carry

    lax.fori_loop(0, CHUNK, zrow, 0)
    zbuf = rows_v.at[0, 0]
    pltpu.sync_copy(b_hbm.at[cid], bvec)
    for k in range(8):
        c = sid + NS * k

        @pl.when(c < RCH)
        def _():
            pltpu.sync_copy(zbuf, acc.at[pl.ds(c * CHUNK, CHUNK)])
    plsc.subcore_barrier()

    hv = h_hbm.at[cid]

    def fetch_idx(g, d, sem):
        pltpu.async_copy(src_hbm.at[sid, g], src_v.at[d], sem)
        pltpu.async_copy(dst_hbm.at[sid, g], dst_v.at[d], sem)
        pltpu.async_copy(adj_hbm.at[sid, g], adj_v.at[d], sem)

    def wait_idx(g, d, sem):
        pltpu.make_async_copy(src_hbm.at[sid, g], src_v.at[d], sem).wait()
        pltpu.make_async_copy(dst_hbm.at[sid, g], dst_v.at[d], sem).wait()
        pltpu.make_async_copy(adj_hbm.at[sid, g], adj_v.at[d], sem).wait()

    def fire_gathers(d, sem):
        for b in range(NBUF):
            pltpu.async_copy(hv.at[src_v.at[d, b]], rows_v.at[d, b], sem)

    def drain_gathers(d, sem):
        for b in range(NBUF):
            pltpu.make_async_copy(hv.at[src_v.at[d, b]],
                                  rows_v.at[d, b], sem).wait()

    def drain_scatters(d, sem):
        for b in range(NBUF):
            pltpu.make_async_copy(rows_v.at[d, b],
                                  acc.at[dst_v.at[d, b]], sem).wait()

    fetch_idx(0, 0, isem[0])
    wait_idx(0, 0, isem[0])
    fire_gathers(0, gsem[0])
    fetch_idx(1, 1, isem[1])

    def dbl_body(gg, carry):
        for d in range(2):
            g = gg * 2 + d
            drain_gathers(d, gsem[d])

            @pl.when(g + 1 < NOUT)
            def _():
                wait_idx(g + 1, 1 - d, isem[1 - d])
                fire_gathers(1 - d, gsem[1 - d])

            for b in range(NBUF):
                _scale_chunk(rows_v, adj_v, d, b)
                pltpu.async_copy(rows_v.at[d, b], acc.at[dst_v.at[d, b]],
                                 ssem[d], add=True)
            drain_scatters(d, ssem[d])

            @pl.when(g + 2 < NOUT)
            def _():
                fetch_idx(g + 2, d, isem[d])
        return carry

    lax.fori_loop(0, NOUT // 2, dbl_body, 0)
    plsc.subcore_barrier()

    for k in range(8):
        c = sid + NS * k

        @pl.when(c < RCH)
        def _():
            rs = pl.ds(c * CHUNK, CHUNK)
            pltpu.sync_copy(acc.at[rs], zbuf)

            def frow(r, carry):
                for j in range(DH // 16):
                    sl = pl.ds(j * 16, 16)
                    zbuf[r, sl] = jnp.maximum(zbuf[r, sl] + bvec[sl], 0.0)
                return carry

            lax.fori_loop(0, CHUNK, frow, 0)
            pltpu.sync_copy(zbuf, out_hbm.at[rs, pl.ds(cid * DH, DH)])


@jax.jit
def _sc_spmm(hh, dst, src, adj_vals, bh):
    mesh = plsc.VectorSubcoreMesh(core_axis_name="c", subcore_axis_name="s")
    return pl.kernel(
        _sc_body,
        out_type=jax.ShapeDtypeStruct((N, D), jnp.float32),
        mesh=mesh,
        compiler_params=pltpu.CompilerParams(use_tc_tiling_on_sc=False),
        scratch_types=[
            pltpu.VMEM((2, NBUF, CHUNK), jnp.int32),
            pltpu.VMEM((2, NBUF, CHUNK), jnp.int32),
            pltpu.VMEM((2, NBUF, CHUNK), jnp.float32),
            pltpu.VMEM((2, NBUF, CHUNK, DH), jnp.float32),
            pltpu.VMEM((DH,), jnp.float32),
            pltpu.VMEM_SHARED((N, DH), jnp.float32),
            pltpu.SemaphoreType.DMA,
            pltpu.SemaphoreType.DMA,
            pltpu.SemaphoreType.DMA,
            pltpu.SemaphoreType.DMA,
            pltpu.SemaphoreType.DMA,
            pltpu.SemaphoreType.DMA,
        ],
    )(hh, dst.reshape(NS, NOUT, NBUF, CHUNK),
      src.reshape(NS, NOUT, NBUF, CHUNK),
      adj_vals.reshape(NS, NOUT, NBUF, CHUNK), bh)


def _tc_body(x_ref, w_ref, o_ref):
    h = jnp.dot(x_ref[...], w_ref[...], preferred_element_type=jnp.float32)
    o_ref[0] = h[:, :DH]
    o_ref[1] = h[:, DH:]


BM = 1000


@jax.jit
def _tc_matmul(x, W):
    return pl.pallas_call(
        _tc_body,
        grid=(N // BM,),
        in_specs=[
            pl.BlockSpec((BM, D), lambda i: (i, 0)),
            pl.BlockSpec((D, D), lambda i: (0, 0)),
        ],
        out_specs=pl.BlockSpec((NC, BM, DH), lambda i: (0, i, 0)),
        out_shape=jax.ShapeDtypeStruct((NC, N, DH), jnp.float32),
    )(x, W)


def kernel(x, edge_index, adj_vals, W, b):
    hh = _tc_matmul(x.astype(jnp.float32), W.astype(jnp.float32))
    bh = b.astype(jnp.float32).reshape(NC, DH)
    dst = edge_index[0]
    src = edge_index[1]
    return _sc_spmm(hh, dst, src, adj_vals, bh)


# speedup vs baseline: 1.2138x; 1.0581x over previous
import jax
import jax.numpy as jnp
from jax import lax
from jax.experimental import pallas as pl
from jax.experimental.pallas import tpu as pltpu
from jax.experimental.pallas import tpu_sc as plsc

N = 10000
E = 320000
D = 128

NC = 2
NS = 16
DH = D // NC
EPW = E // NS
CHUNK = 80
NBUF = 5
BLK = NBUF * CHUNK
NOUT = EPW // BLK
RCH = N // CHUNK


def _scale_chunk(rows_v, adj_v, d, b):
    for k in range(CHUNK // 8):
        base = min(8 * k, CHUNK - 16)
        loff = 8 * k - base
        a16 = adj_v[d, b, pl.ds(base, 16)]
        scaled = []
        for i in range(8):
            e = 8 * k + i
            a = jnp.full((16,), a16[loff + i], jnp.float32)
            for j in range(DH // 16):
                scaled.append(rows_v[d, b, e, pl.ds(j * 16, 16)] * a)
        for i in range(8):
            e = 8 * k + i
            for j in range(DH // 16):
                rows_v[d, b, e, pl.ds(j * 16, 16)] = scaled[i * (DH // 16) + j]


def _sc_body(h_hbm, dst_hbm, src_hbm, adj_hbm, b_hbm, out_hbm,
             src_v, dst_v, adj_v, rows_v, bvec, acc,
             gs0, gs1, is0, is1, ss0, ss1, ds0, ds1):
    cid = lax.axis_index("c")
    sid = lax.axis_index("s")
    gsem = (gs0, gs1)
    isem = (is0, is1)
    ssem = (ss0, ss1)
    dsem = (ds0, ds1)

    zero16 = jnp.zeros((16,), jnp.float32)

    def zrow(i, carry):
        for j in range(DH // 16):
            rows_v[0, 0, i, pl.ds(j * 16, 16)] = zero16
        return carry

    lax.fori_loop(0, CHUNK, zrow, 0)
    zbuf = rows_v.at[0, 0]
    pltpu.sync_copy(b_hbm.at[cid], bvec)
    for k in range(8):
        c = sid + NS * k

        @pl.when(c < RCH)
        def _():
            pltpu.sync_copy(zbuf, acc.at[pl.ds(c * CHUNK, CHUNK)])
    plsc.subcore_barrier()

    hv = h_hbm.at[cid]

    def fetch_idx(g, d, sem):
        pltpu.async_copy(src_hbm.at[sid, g], src_v.at[d], sem)
        pltpu.async_copy(adj_hbm.at[sid, g], adj_v.at[d], sem)

    def wait_idx(g, d, sem):
        pltpu.make_async_copy(src_hbm.at[sid, g], src_v.at[d], sem).wait()
        pltpu.make_async_copy(adj_hbm.at[sid, g], adj_v.at[d], sem).wait()

    def fetch_dst(g, d, sem):
        pltpu.async_copy(dst_hbm.at[sid, g], dst_v.at[d], sem)

    def wait_dst(g, d, sem):
        pltpu.make_async_copy(dst_hbm.at[sid, g], dst_v.at[d], sem).wait()

    def fire_gathers(d, sem):
        for b in range(NBUF):
            pltpu.async_copy(hv.at[src_v.at[d, b]], rows_v.at[d, b], sem)

    def drain_gathers(d, sem):
        for b in range(NBUF):
            pltpu.make_async_copy(hv.at[src_v.at[d, b]],
                                  rows_v.at[d, b], sem).wait()

    def drain_scatters(d, sem):
        for b in range(NBUF):
            pltpu.make_async_copy(rows_v.at[d, b],
                                  acc.at[dst_v.at[d, b]], sem).wait()

    fetch_idx(0, 0, isem[0])
    wait_idx(0, 0, isem[0])
    fetch_dst(0, 0, dsem[0])
    fetch_dst(1, 1, dsem[1])
    fire_gathers(0, gsem[0])
    fetch_idx(1, 1, isem[1])

    def dbl_body(gg, carry):
        for d in range(2):
            g = gg * 2 + d
            drain_gathers(d, gsem[d])

            @pl.when(g + 1 < NOUT)
            def _():
                wait_idx(g + 1, 1 - d, isem[1 - d])

                @pl.when(g >= 1)
                def _():
                    drain_scatters(1 - d, ssem[1 - d])
                    fetch_dst(g + 1, 1 - d, dsem[1 - d])

                fire_gathers(1 - d, gsem[1 - d])

            wait_dst(g, d, dsem[d])
            for b in range(NBUF):
                _scale_chunk(rows_v, adj_v, d, b)
                pltpu.async_copy(rows_v.at[d, b], acc.at[dst_v.at[d, b]],
                                 ssem[d], add=True)

            @pl.when(g + 2 < NOUT)
            def _():
                fetch_idx(g + 2, d, isem[d])
        return carry

    lax.fori_loop(0, NOUT // 2, dbl_body, 0)
    drain_scatters(0, ssem[0])
    drain_scatters(1, ssem[1])
    plsc.subcore_barrier()

    for k in range(8):
        c = sid + NS * k

        @pl.when(c < RCH)
        def _():
            rs = pl.ds(c * CHUNK, CHUNK)
            pltpu.sync_copy(acc.at[rs], zbuf)

            def frow(r, carry):
                for j in range(DH // 16):
                    sl = pl.ds(j * 16, 16)
                    zbuf[r, sl] = jnp.maximum(zbuf[r, sl] + bvec[sl], 0.0)
                return carry

            lax.fori_loop(0, CHUNK, frow, 0)
            pltpu.sync_copy(zbuf, out_hbm.at[rs, pl.ds(cid * DH, DH)])


@jax.jit
def _sc_spmm(hh, dst, src, adj_vals, bh):
    mesh = plsc.VectorSubcoreMesh(core_axis_name="c", subcore_axis_name="s")
    return pl.kernel(
        _sc_body,
        out_type=jax.ShapeDtypeStruct((N, D), jnp.float32),
        mesh=mesh,
        compiler_params=pltpu.CompilerParams(use_tc_tiling_on_sc=False),
        scratch_types=[
            pltpu.VMEM((2, NBUF, CHUNK), jnp.int32),
            pltpu.VMEM((2, NBUF, CHUNK), jnp.int32),
            pltpu.VMEM((2, NBUF, CHUNK), jnp.float32),
            pltpu.VMEM((2, NBUF, CHUNK, DH), jnp.float32),
            pltpu.VMEM((DH,), jnp.float32),
            pltpu.VMEM_SHARED((N, DH), jnp.float32),
            pltpu.SemaphoreType.DMA,
            pltpu.SemaphoreType.DMA,
            pltpu.SemaphoreType.DMA,
            pltpu.SemaphoreType.DMA,
            pltpu.SemaphoreType.DMA,
            pltpu.SemaphoreType.DMA,
            pltpu.SemaphoreType.DMA,
            pltpu.SemaphoreType.DMA,
        ],
    )(hh, dst.reshape(NS, NOUT, NBUF, CHUNK),
      src.reshape(NS, NOUT, NBUF, CHUNK),
      adj_vals.reshape(NS, NOUT, NBUF, CHUNK), bh)


def _tc_body(x_ref, w_ref, o_ref):
    h = jnp.dot(x_ref[...], w_ref[...], preferred_element_type=jnp.float32)
    o_ref[0] = h[:, :DH]
    o_ref[1] = h[:, DH:]


BM = 1000


@jax.jit
def _tc_matmul(x, W):
    return pl.pallas_call(
        _tc_body,
        grid=(N // BM,),
        in_specs=[
            pl.BlockSpec((BM, D), lambda i: (i, 0)),
            pl.BlockSpec((D, D), lambda i: (0, 0)),
        ],
        out_specs=pl.BlockSpec((NC, BM, DH), lambda i: (0, i, 0)),
        out_shape=jax.ShapeDtypeStruct((NC, N, DH), jnp.float32),
    )(x, W)


def kernel(x, edge_index, adj_vals, W, b):
    hh = _tc_matmul(x.astype(jnp.float32), W.astype(jnp.float32))
    bh = b.astype(jnp.float32).reshape(NC, DH)
    dst = edge_index[0]
    src = edge_index[1]
    return _sc_spmm(hh, dst, src, adj_vals, bh)
